# Initial kernel scaffold; baseline (speedup 1.0000x reference)
import jax
import jax.numpy as jnp
from jax.experimental import pallas as pl


def kernel(x, edge_index, edge_weight, W0, b0, W1, b1, W2, b2, Wl, bl):
    raise NotImplementedError("write your pallas kernel here")


# speedup vs baseline: 3.3232x; 3.3232x over previous
import functools

import jax
import jax.numpy as jnp
from jax import lax
from jax.experimental import pallas as pl
from jax.experimental.pallas import tpu as pltpu
from jax.experimental.pallas import tpu_sc as plsc

N = 10000
E = 160000
D_IN = 256
D_HID = 512
D_OUT = 256

CHUNK = 128
NCHUNK = D_HID // CHUNK
NC = 2
NS = 16
CHUNKS_PER_CORE = NCHUNK // NC
EPT = E // NS
KB = 80
NBATCH = EPT // KB
NDRAIN = 10
RPT = N // NDRAIN
ZROWS = 40
MB = 1000


def _mm_first_body(x_ref, w_ref, b_ref, o_ref):
    acc = lax.dot_general(x_ref[...], w_ref[...], (((1,), (1,)), ((), ())),
                          preferred_element_type=jnp.float32)
    o_ref[0] = acc + b_ref[0, 0][None, :]


def _mm_first(x, W, b4):
    return pl.pallas_call(
        _mm_first_body,
        grid=(N // MB, NCHUNK),
        in_specs=[
            pl.BlockSpec((MB, D_IN), lambda m, c: (m, 0)),
            pl.BlockSpec((CHUNK, D_IN), lambda m, c: (c, 0)),
            pl.BlockSpec((1, 1, CHUNK), lambda m, c: (c, 0, 0)),
        ],
        out_specs=pl.BlockSpec((1, MB, CHUNK), lambda m, c: (c, m, 0)),
        out_shape=jax.ShapeDtypeStruct((NCHUNK, N, CHUNK), jnp.float32),
    )(x, W, b4)


def _mm_mid_body(y_ref, w_ref, b_ref, o_ref):
    acc = jnp.zeros((MB, CHUNK), jnp.float32)
    for kc in range(NCHUNK):
        yk = jnp.maximum(y_ref[kc], 0.0)
        wk = w_ref[:, kc * CHUNK:(kc + 1) * CHUNK]
        acc = acc + lax.dot_general(yk, wk, (((1,), (1,)), ((), ())),
                                    preferred_element_type=jnp.float32)
    o_ref[0] = acc + b_ref[0, 0][None, :]


def _mm_mid(y, W, b4):
    return pl.pallas_call(
        _mm_mid_body,
        grid=(N // MB, NCHUNK),
        in_specs=[
            pl.BlockSpec((NCHUNK, MB, CHUNK), lambda m, c: (0, m, 0)),
            pl.BlockSpec((CHUNK, D_HID), lambda m, c: (c, 0)),
            pl.BlockSpec((1, 1, CHUNK), lambda m, c: (c, 0, 0)),
        ],
        out_specs=pl.BlockSpec((1, MB, CHUNK), lambda m, c: (c, m, 0)),
        out_shape=jax.ShapeDtypeStruct((NCHUNK, N, CHUNK), jnp.float32),
    )(y, W, b4)


def _mm_last_body(y_ref, w_ref, b_ref, o_ref):
    acc = jnp.zeros((MB, D_OUT), jnp.float32)
    for kc in range(NCHUNK):
        yk = jnp.maximum(y_ref[kc], 0.0)
        wk = w_ref[:, kc * CHUNK:(kc + 1) * CHUNK]
        acc = acc + lax.dot_general(yk, wk, (((1,), (1,)), ((), ())),
                                    preferred_element_type=jnp.float32)
    h = acc + b_ref[...]
    norm = jnp.sqrt(jnp.sum(h * h, axis=1, keepdims=True))
    o_ref[...] = h / jnp.maximum(norm, 1e-12)


def _mm_last(y, Wl, bl2):
    return pl.pallas_call(
        _mm_last_body,
        grid=(N // MB,),
        in_specs=[
            pl.BlockSpec((NCHUNK, MB, CHUNK), lambda m: (0, m, 0)),
            pl.BlockSpec((D_OUT, D_HID), lambda m: (0, 0)),
            pl.BlockSpec((1, D_OUT), lambda m: (0, 0)),
        ],
        out_specs=pl.BlockSpec((MB, D_OUT), lambda m: (m, 0)),
        out_shape=jax.ShapeDtypeStruct((N, D_OUT), jnp.float32),
    )(y, Wl, bl2)


_SC_MESH = plsc.VectorSubcoreMesh(core_axis_name="c", subcore_axis_name="s")


@functools.partial(
    pl.kernel,
    out_type=jax.ShapeDtypeStruct((NCHUNK, N, CHUNK), jnp.float32),
    mesh=_SC_MESH,
    scratch_types=[
        pltpu.VMEM((EPT,), jnp.int32),
        pltpu.VMEM((EPT,), jnp.int32),
        pltpu.VMEM((EPT,), jnp.float32),
        pltpu.VMEM((KB,), jnp.int32),
        pltpu.VMEM((KB, CHUNK), jnp.float32),
        pltpu.VMEM((ZROWS, CHUNK), jnp.float32),
        pltpu.VMEM_SHARED((N, CHUNK), jnp.float32),
        pltpu.SemaphoreType.DMA,
    ],
)
def _sc_scatter(h_hbm, src_hbm, dst_hbm, w_hbm, out_hbm,
                src_v, dst_v, w_v, dstb_v, rows_v, zero_v,
                acc_sh, sem):
    cid = lax.axis_index("c")
    sid = lax.axis_index("s")
    e0 = pl.multiple_of(sid * EPT, 8)
    r0 = pl.multiple_of(sid * RPT, 8)

    pltpu.sync_copy(src_hbm.at[pl.ds(e0, EPT)], src_v)
    pltpu.sync_copy(dst_hbm.at[pl.ds(e0, EPT)], dst_v)
    pltpu.sync_copy(w_hbm.at[pl.ds(e0, EPT)], w_v)

    def _zrow(r, carry):
        for f in range(CHUNK // 16):
            zero_v[r, pl.ds(f * 16, 16)] = jnp.zeros((16,), jnp.float32)
        return carry
    lax.fori_loop(0, ZROWS, _zrow, 0)

    for i in range(CHUNKS_PER_CORE):
        c = cid * CHUNKS_PER_CORE + i

        @pl.when(sid < NDRAIN)
        def _zero():
            for q in range(RPT // ZROWS):
                pltpu.sync_copy(zero_v,
                                acc_sh.at[pl.ds(r0 + q * ZROWS, ZROWS)])
        plsc.subcore_barrier()

        def _batch(b, carry):
            off = pl.multiple_of(b * KB, 8)
            pltpu.async_copy(h_hbm.at[c].at[src_v.at[pl.ds(off, KB)]],
                             rows_v, sem).wait()
            for q in range(KB // 16):
                dstb_v[pl.ds(q * 16, 16)] = dst_v[pl.ds(off + q * 16, 16)]

            def _grp(g, carry2):
                w16 = w_v[pl.ds(off + g * 16, 16)]
                for j2 in range(16):
                    wv = jnp.full((16,), w16[j2], jnp.float32)
                    r = g * 16 + j2
                    for f in range(CHUNK // 16):
                        sl = pl.ds(f * 16, 16)
                        rows_v[r, sl] = rows_v[r, sl] * wv
                return carry2
            lax.fori_loop(0, KB // 16, _grp, 0)

            pltpu.sync_copy(rows_v, acc_sh.at[dstb_v], add=True)
            return carry
        lax.fori_loop(0, NBATCH, _batch, 0)
        plsc.subcore_barrier()

        @pl.when(sid < NDRAIN)
        def _drain():
            for q in range(RPT // ZROWS):
                pltpu.sync_copy(acc_sh.at[pl.ds(r0 + q * ZROWS, ZROWS)],
                                rows_v.at[pl.ds(0, ZROWS)])
                pltpu.sync_copy(rows_v.at[pl.ds(0, ZROWS)],
                                out_hbm.at[c, pl.ds(r0 + q * ZROWS, ZROWS)])


def kernel(x, edge_index, edge_weight, W0, b0, W1, b1, W2, b2, Wl, bl):
    dst = edge_index[0]
    src = edge_index[1]

    h = _mm_first(x, W0, b0.reshape(NCHUNK, 1, CHUNK))
    y = _sc_scatter(h, src, dst, edge_weight)
    h = _mm_mid(y, W1, b1.reshape(NCHUNK, 1, CHUNK))
    y = _sc_scatter(h, src, dst, edge_weight)
    h = _mm_mid(y, W2, b2.reshape(NCHUNK, 1, CHUNK))
    y = _sc_scatter(h, src, dst, edge_weight)
    return _mm_last(y, Wl, bl.reshape(1, D_OUT))


# speedup vs baseline: 6.6279x; 1.9944x over previous
import functools

import jax
import jax.numpy as jnp
from jax import lax
from jax.experimental import pallas as pl
from jax.experimental.pallas import tpu as pltpu
from jax.experimental.pallas import tpu_sc as plsc

N = 10000
E = 160000
D_IN = 256
D_HID = 512
D_OUT = 256

CHUNK = 128
NCHUNK = D_HID // CHUNK
NC = 2
NS = 16
CHUNKS_PER_CORE = NCHUNK // NC
EPT = E // NS
KB = 80
NBATCH = EPT // KB
RING = 4
LOOK = 2
NB_MAIN = NBATCH - 1
ZROWS = 40
RPT = 640
RPT_LAST = N - 15 * RPT
MB = 1000


def _mm_first_body(x_ref, w_ref, b_ref, o_ref):
    acc = lax.dot_general(x_ref[...], w_ref[...], (((1,), (1,)), ((), ())),
                          preferred_element_type=jnp.float32)
    o_ref[0] = acc + b_ref[0, 0][None, :]


def _mm_first(x, W, b4):
    return pl.pallas_call(
        _mm_first_body,
        grid=(N // MB, NCHUNK),
        in_specs=[
            pl.BlockSpec((MB, D_IN), lambda m, c: (m, 0)),
            pl.BlockSpec((CHUNK, D_IN), lambda m, c: (c, 0)),
            pl.BlockSpec((1, 1, CHUNK), lambda m, c: (c, 0, 0)),
        ],
        out_specs=pl.BlockSpec((1, MB, CHUNK), lambda m, c: (c, m, 0)),
        out_shape=jax.ShapeDtypeStruct((NCHUNK, N, CHUNK), jnp.float32),
    )(x, W, b4)


def _mm_mid_body(y_ref, w_ref, b_ref, o_ref):
    acc = jnp.zeros((MB, CHUNK), jnp.float32)
    for kc in range(NCHUNK):
        yk = jnp.maximum(y_ref[kc], 0.0)
        wk = w_ref[:, kc * CHUNK:(kc + 1) * CHUNK]
        acc = acc + lax.dot_general(yk, wk, (((1,), (1,)), ((), ())),
                                    preferred_element_type=jnp.float32)
    o_ref[0] = acc + b_ref[0, 0][None, :]


def _mm_mid(y, W, b4):
    return pl.pallas_call(
        _mm_mid_body,
        grid=(N // MB, NCHUNK),
        in_specs=[
            pl.BlockSpec((NCHUNK, MB, CHUNK), lambda m, c: (0, m, 0)),
            pl.BlockSpec((CHUNK, D_HID), lambda m, c: (c, 0)),
            pl.BlockSpec((1, 1, CHUNK), lambda m, c: (c, 0, 0)),
        ],
        out_specs=pl.BlockSpec((1, MB, CHUNK), lambda m, c: (c, m, 0)),
        out_shape=jax.ShapeDtypeStruct((NCHUNK, N, CHUNK), jnp.float32),
    )(y, W, b4)


def _mm_last_body(y_ref, w_ref, b_ref, o_ref):
    acc = jnp.zeros((MB, D_OUT), jnp.float32)
    for kc in range(NCHUNK):
        yk = jnp.maximum(y_ref[kc], 0.0)
        wk = w_ref[:, kc * CHUNK:(kc + 1) * CHUNK]
        acc = acc + lax.dot_general(yk, wk, (((1,), (1,)), ((), ())),
                                    preferred_element_type=jnp.float32)
    h = acc + b_ref[...]
    norm = jnp.sqrt(jnp.sum(h * h, axis=1, keepdims=True))
    o_ref[...] = h / jnp.maximum(norm, 1e-12)


def _mm_last(y, Wl, bl2):
    return pl.pallas_call(
        _mm_last_body,
        grid=(N // MB,),
        in_specs=[
            pl.BlockSpec((NCHUNK, MB, CHUNK), lambda m: (0, m, 0)),
            pl.BlockSpec((D_OUT, D_HID), lambda m: (0, 0)),
            pl.BlockSpec((1, D_OUT), lambda m: (0, 0)),
        ],
        out_specs=pl.BlockSpec((MB, D_OUT), lambda m: (m, 0)),
        out_shape=jax.ShapeDtypeStruct((N, D_OUT), jnp.float32),
    )(y, Wl, bl2)


_SC_MESH = plsc.VectorSubcoreMesh(core_axis_name="c", subcore_axis_name="s")


_SC_SCRATCH = (
    [pltpu.VMEM((KB,), jnp.int32) for _ in range(RING)]
    + [pltpu.VMEM((KB,), jnp.int32) for _ in range(RING)]
    + [pltpu.VMEM((KB,), jnp.float32) for _ in range(RING)]
    + [pltpu.VMEM((KB, CHUNK), jnp.float32) for _ in range(RING)]
    + [pltpu.VMEM((ZROWS, CHUNK), jnp.float32)]
    + [pltpu.VMEM_SHARED((N, CHUNK), jnp.float32)]
    + [pltpu.SemaphoreType.DMA for _ in range(4 * RING + 1)]
)


@functools.partial(
    pl.kernel,
    out_type=jax.ShapeDtypeStruct((NCHUNK, N, CHUNK), jnp.float32),
    mesh=_SC_MESH,
    scratch_types=_SC_SCRATCH,
)
def _sc_scatter(h_hbm, src_hbm, dst_hbm, w_hbm, out_hbm, *scr):
    srcb = list(scr[0:RING])
    dstb = list(scr[RING:2 * RING])
    wb = list(scr[2 * RING:3 * RING])
    rows = list(scr[3 * RING:4 * RING])
    zero_v = scr[4 * RING]
    acc_sh = scr[4 * RING + 1]
    semI = list(scr[4 * RING + 2:4 * RING + 2 + RING])
    semD = list(scr[4 * RING + 2 + RING:4 * RING + 2 + 2 * RING])
    semG = list(scr[4 * RING + 2 + 2 * RING:4 * RING + 2 + 3 * RING])
    semS = list(scr[4 * RING + 2 + 3 * RING:4 * RING + 2 + 4 * RING])
    semZ = scr[4 * RING + 2 + 4 * RING]

    cid = lax.axis_index("c")
    sid = lax.axis_index("s")
    e0 = pl.multiple_of(sid * EPT, 8)
    r0 = pl.multiple_of(sid * RPT, 8)

    def _fetch_srcw(b, k):
        off = e0 + pl.multiple_of(b * KB, 8)
        pltpu.async_copy(src_hbm.at[pl.ds(off, KB)], srcb[k], semI[k])
        pltpu.async_copy(w_hbm.at[pl.ds(off, KB)], wb[k], semI[k])

    def _wait_srcw(k):
        pltpu.make_async_copy(src_hbm.at[pl.ds(e0, KB)], srcb[k],
                              semI[k]).wait()
        pltpu.make_async_copy(w_hbm.at[pl.ds(e0, KB)], wb[k],
                              semI[k]).wait()

    def _fetch_dst(b, k):
        off = e0 + pl.multiple_of(b * KB, 8)
        pltpu.async_copy(dst_hbm.at[pl.ds(off, KB)], dstb[k], semD[k])

    def _wait_dst(k):
        pltpu.make_async_copy(dst_hbm.at[pl.ds(e0, KB)], dstb[k],
                              semD[k]).wait()

    def _gather(c, k):
        pltpu.async_copy(h_hbm.at[c].at[srcb[k]], rows[k], semG[k])

    def _wait_gather(c, k):
        pltpu.make_async_copy(h_hbm.at[c].at[srcb[k]], rows[k],
                              semG[k]).wait()

    def _scatter(k):
        pltpu.async_copy(rows[k], acc_sh.at[dstb[k]], semS[k], add=True)

    def _wait_scatter(k):
        pltpu.make_async_copy(rows[k], acc_sh.at[dstb[k]], semS[k]).wait()

    def _visit(b, k, c, refill):
        _wait_gather(c, k)

        def _grp(g, carry2):
            w16 = wb[k][pl.ds(g * 16, 16)]
            for j2 in range(16):
                wv = jnp.full((16,), w16[j2], jnp.float32)
                r = g * 16 + j2
                for f in range(CHUNK // 16):
                    sl = pl.ds(f * 16, 16)
                    rows[k][r, sl] = rows[k][r, sl] * wv
            return carry2
        lax.fori_loop(0, KB // 16, _grp, 0)

        _wait_dst(k)
        _scatter(k)

        if refill:
            k2 = (k + LOOK) % RING
            bt = b + LOOK

            @pl.when(bt < NBATCH)
            def _refill():
                @pl.when(b >= LOOK)
                def _protect():
                    _wait_scatter(k2)
                _fetch_dst(bt, k2)
                _wait_srcw(k2)
                _gather(c, k2)

            @pl.when(b + RING < NBATCH)
            def _prefetch():
                _fetch_srcw(b + RING, k)

    for r in range(ZROWS):
        for f in range(CHUNK // 16):
            zero_v[r, pl.ds(f * 16, 16)] = jnp.zeros((16,), jnp.float32)

    for i in range(CHUNKS_PER_CORE):
        c = cid * CHUNKS_PER_CORE + i

        for k in range(RING):
            _fetch_srcw(k, k)
        for k in range(LOOK):
            _fetch_dst(k, k)
        for k in range(LOOK):
            _wait_srcw(k)
            _gather(c, k)

        @pl.when(sid < NS - 1)
        def _zero():
            for q in range(RPT // ZROWS):
                pltpu.async_copy(zero_v,
                                 acc_sh.at[pl.ds(r0 + q * ZROWS, ZROWS)],
                                 semZ)
            for q in range(RPT // ZROWS):
                pltpu.make_async_copy(zero_v,
                                      acc_sh.at[pl.ds(r0, ZROWS)],
                                      semZ).wait()

        @pl.when(sid == NS - 1)
        def _zero_last():
            for q in range(RPT_LAST // ZROWS):
                pltpu.async_copy(zero_v,
                                 acc_sh.at[pl.ds(r0 + q * ZROWS, ZROWS)],
                                 semZ)
            for q in range(RPT_LAST // ZROWS):
                pltpu.make_async_copy(zero_v,
                                      acc_sh.at[pl.ds(r0, ZROWS)],
                                      semZ).wait()
        plsc.subcore_barrier()

        def _iter(it, carry):
            b_base = it * RING
            for k in range(RING):
                _visit(b_base + k, k, c, refill=True)
            return carry
        lax.fori_loop(0, NB_MAIN // RING, _iter, 0)

        _visit(NB_MAIN, NB_MAIN % RING, c, refill=False)

        for k in range(RING):
            _wait_scatter(k)
        plsc.subcore_barrier()

        @pl.when(sid < NS - 1)
        def _drain():
            pltpu.sync_copy(acc_sh.at[pl.ds(r0, RPT)],
                            out_hbm.at[c, pl.ds(r0, RPT)])

        @pl.when(sid == NS - 1)
        def _drain_last():
            pltpu.sync_copy(acc_sh.at[pl.ds(r0, RPT_LAST)],
                            out_hbm.at[c, pl.ds(r0, RPT_LAST)])


def kernel(x, edge_index, edge_weight, W0, b0, W1, b1, W2, b2, Wl, bl):
    dst = edge_index[0]
    src = edge_index[1]

    h = _mm_first(x, W0, b0.reshape(NCHUNK, 1, CHUNK))
    y = _sc_scatter(h, src, dst, edge_weight)
    h = _mm_mid(y, W1, b1.reshape(NCHUNK, 1, CHUNK))
    y = _sc_scatter(h, src, dst, edge_weight)
    h = _mm_mid(y, W2, b2.reshape(NCHUNK, 1, CHUNK))
    y = _sc_scatter(h, src, dst, edge_weight)
    return _mm_last(y, Wl, bl.reshape(1, D_OUT))


# speedup vs baseline: 6.6484x; 1.0031x over previous
import functools

import jax
import jax.numpy as jnp
from jax import lax
from jax.experimental import pallas as pl
from jax.experimental.pallas import tpu as pltpu
from jax.experimental.pallas import tpu_sc as plsc

N = 10000
E = 160000
D_IN = 256
D_HID = 512
D_OUT = 256

CHUNK = 128
NCHUNK = D_HID // CHUNK
NC = 2
NS = 16
CHUNKS_PER_CORE = NCHUNK // NC
EPT = E // NS
KB = 80
NBATCH = EPT // KB
RING = 4
LOOK = 2
NB_MAIN = NBATCH - 1
ZROWS = 40
RPT = 640
RPT_LAST = N - 15 * RPT
MB = 1000


def _mm_first_body(x_ref, w_ref, b_ref, o_ref):
    acc = lax.dot_general(x_ref[...].astype(jnp.bfloat16), w_ref[...],
                          (((1,), (1,)), ((), ())),
                          preferred_element_type=jnp.float32)
    o_ref[0] = acc + b_ref[0, 0][None, :]


def _mm_first(x, W, b4):
    return pl.pallas_call(
        _mm_first_body,
        grid=(N // MB, NCHUNK),
        in_specs=[
            pl.BlockSpec((MB, D_IN), lambda m, c: (m, 0)),
            pl.BlockSpec((CHUNK, D_IN), lambda m, c: (c, 0)),
            pl.BlockSpec((1, 1, CHUNK), lambda m, c: (c, 0, 0)),
        ],
        out_specs=pl.BlockSpec((1, MB, CHUNK), lambda m, c: (c, m, 0)),
        out_shape=jax.ShapeDtypeStruct((NCHUNK, N, CHUNK), jnp.float32),
    )(x, W, b4)


def _mm_mid_body(y_ref, w_ref, b_ref, o_ref):
    acc = jnp.zeros((MB, CHUNK), jnp.float32)
    for kc in range(NCHUNK):
        yk = jnp.maximum(y_ref[kc], 0.0).astype(jnp.bfloat16)
        wk = w_ref[:, kc * CHUNK:(kc + 1) * CHUNK]
        acc = acc + lax.dot_general(yk, wk, (((1,), (1,)), ((), ())),
                                    preferred_element_type=jnp.float32)
    o_ref[0] = acc + b_ref[0, 0][None, :]


def _mm_mid(y, W, b4):
    return pl.pallas_call(
        _mm_mid_body,
        grid=(N // MB, NCHUNK),
        in_specs=[
            pl.BlockSpec((NCHUNK, MB, CHUNK), lambda m, c: (0, m, 0)),
            pl.BlockSpec((CHUNK, D_HID), lambda m, c: (c, 0)),
            pl.BlockSpec((1, 1, CHUNK), lambda m, c: (c, 0, 0)),
        ],
        out_specs=pl.BlockSpec((1, MB, CHUNK), lambda m, c: (c, m, 0)),
        out_shape=jax.ShapeDtypeStruct((NCHUNK, N, CHUNK), jnp.float32),
    )(y, W, b4)


def _mm_last_body(y_ref, w_ref, b_ref, o_ref):
    acc = jnp.zeros((MB, D_OUT), jnp.float32)
    for kc in range(NCHUNK):
        yk = jnp.maximum(y_ref[kc], 0.0).astype(jnp.bfloat16)
        wk = w_ref[:, kc * CHUNK:(kc + 1) * CHUNK]
        acc = acc + lax.dot_general(yk, wk, (((1,), (1,)), ((), ())),
                                    preferred_element_type=jnp.float32)
    h = acc + b_ref[...]
    norm = jnp.sqrt(jnp.sum(h * h, axis=1, keepdims=True))
    o_ref[...] = h / jnp.maximum(norm, 1e-12)


def _mm_last(y, Wl, bl2):
    return pl.pallas_call(
        _mm_last_body,
        grid=(N // MB,),
        in_specs=[
            pl.BlockSpec((NCHUNK, MB, CHUNK), lambda m: (0, m, 0)),
            pl.BlockSpec((D_OUT, D_HID), lambda m: (0, 0)),
            pl.BlockSpec((1, D_OUT), lambda m: (0, 0)),
        ],
        out_specs=pl.BlockSpec((MB, D_OUT), lambda m: (m, 0)),
        out_shape=jax.ShapeDtypeStruct((N, D_OUT), jnp.float32),
    )(y, Wl, bl2)


_SC_MESH = plsc.VectorSubcoreMesh(core_axis_name="c", subcore_axis_name="s")


_SC_SCRATCH = (
    [pltpu.VMEM((KB,), jnp.int32) for _ in range(RING)]
    + [pltpu.VMEM((KB,), jnp.int32) for _ in range(RING)]
    + [pltpu.VMEM((KB,), jnp.float32) for _ in range(RING)]
    + [pltpu.VMEM((KB, CHUNK), jnp.float32) for _ in range(RING)]
    + [pltpu.VMEM((ZROWS, CHUNK), jnp.float32)]
    + [pltpu.VMEM_SHARED((N, CHUNK), jnp.float32)]
    + [pltpu.SemaphoreType.DMA for _ in range(4 * RING + 1)]
)


@functools.partial(
    pl.kernel,
    out_type=jax.ShapeDtypeStruct((NCHUNK, N, CHUNK), jnp.float32),
    mesh=_SC_MESH,
    scratch_types=_SC_SCRATCH,
)
def _sc_scatter(h_hbm, src_hbm, dst_hbm, w_hbm, out_hbm, *scr):
    it = iter(scr)
    srcb = [next(it) for _ in range(RING)]
    dstb = [next(it) for _ in range(RING)]
    wb = [next(it) for _ in range(RING)]
    rows = [next(it) for _ in range(RING)]
    zero_v = next(it)
    acc_sh = next(it)
    semI = [next(it) for _ in range(RING)]
    semD = [next(it) for _ in range(RING)]
    semG = [next(it) for _ in range(RING)]
    semS = [next(it) for _ in range(RING)]
    semZ = next(it)

    cid = lax.axis_index("c")
    sid = lax.axis_index("s")
    e0 = pl.multiple_of(sid * EPT, 8)
    r0 = pl.multiple_of(sid * RPT, 8)

    def _fetch_srcw(b, k):
        off = e0 + pl.multiple_of(b * KB, 8)
        pltpu.async_copy(src_hbm.at[pl.ds(off, KB)], srcb[k], semI[k])
        pltpu.async_copy(w_hbm.at[pl.ds(off, KB)], wb[k], semI[k])

    def _wait_srcw(k):
        pltpu.make_async_copy(src_hbm.at[pl.ds(e0, KB)], srcb[k],
                              semI[k]).wait()
        pltpu.make_async_copy(w_hbm.at[pl.ds(e0, KB)], wb[k],
                              semI[k]).wait()

    def _fetch_dst(b, k):
        off = e0 + pl.multiple_of(b * KB, 8)
        pltpu.async_copy(dst_hbm.at[pl.ds(off, KB)], dstb[k], semD[k])

    def _wait_dst(k):
        pltpu.make_async_copy(dst_hbm.at[pl.ds(e0, KB)], dstb[k],
                              semD[k]).wait()

    def _gather(c, k):
        pltpu.async_copy(h_hbm.at[c].at[srcb[k]], rows[k], semG[k])

    def _wait_gather(c, k):
        pltpu.make_async_copy(h_hbm.at[c].at[srcb[k]], rows[k],
                              semG[k]).wait()

    def _scatter(k):
        pltpu.async_copy(rows[k], acc_sh.at[dstb[k]], semS[k], add=True)

    def _wait_scatter(k):
        pltpu.make_async_copy(rows[k], acc_sh.at[dstb[k]], semS[k]).wait()

    def _visit(b, k, c, refill):
        _wait_gather(c, k)

        def _grp(g, carry2):
            w16 = wb[k][pl.ds(g * 16, 16)]
            for j2 in range(16):
                wv = jnp.full((16,), w16[j2], jnp.float32)
                r = g * 16 + j2
                for f in range(CHUNK // 16):
                    sl = pl.ds(f * 16, 16)
                    rows[k][r, sl] = rows[k][r, sl] * wv
            return carry2
        lax.fori_loop(0, KB // 16, _grp, 0)

        _wait_dst(k)
        _scatter(k)

        if refill:
            k2 = (k + LOOK) % RING
            bt = b + LOOK

            @pl.when(bt < NBATCH)
            def _refill():
                @pl.when(b >= LOOK)
                def _protect():
                    _wait_scatter(k2)
                _fetch_dst(bt, k2)
                _wait_srcw(k2)
                _gather(c, k2)

            @pl.when(b + RING < NBATCH)
            def _prefetch():
                _fetch_srcw(b + RING, k)

    for r in range(ZROWS):
        for f in range(CHUNK // 16):
            zero_v[r, pl.ds(f * 16, 16)] = jnp.zeros((16,), jnp.float32)

    for i in range(CHUNKS_PER_CORE):
        c = cid * CHUNKS_PER_CORE + i

        for k in range(RING):
            _fetch_srcw(k, k)
        for k in range(LOOK):
            _fetch_dst(k, k)
        for k in range(LOOK):
            _wait_srcw(k)
            _gather(c, k)

        @pl.when(sid < NS - 1)
        def _zero():
            for q in range(RPT // ZROWS):
                pltpu.async_copy(zero_v,
                                 acc_sh.at[pl.ds(r0 + q * ZROWS, ZROWS)],
                                 semZ)
            for q in range(RPT // ZROWS):
                pltpu.make_async_copy(zero_v,
                                      acc_sh.at[pl.ds(r0, ZROWS)],
                                      semZ).wait()

        @pl.when(sid == NS - 1)
        def _zero_last():
            for q in range(RPT_LAST // ZROWS):
                pltpu.async_copy(zero_v,
                                 acc_sh.at[pl.ds(r0 + q * ZROWS, ZROWS)],
                                 semZ)
            for q in range(RPT_LAST // ZROWS):
                pltpu.make_async_copy(zero_v,
                                      acc_sh.at[pl.ds(r0, ZROWS)],
                                      semZ).wait()
        plsc.subcore_barrier()

        def _iter(it, carry):
            b_base = it * RING
            for k in range(RING):
                _visit(b_base + k, k, c, refill=True)
            return carry
        lax.fori_loop(0, NB_MAIN // RING, _iter, 0)

        _visit(NB_MAIN, NB_MAIN % RING, c, refill=False)

        for k in range(RING):
            _wait_scatter(k)
        plsc.subcore_barrier()

        @pl.when(sid < NS - 1)
        def _drain():
            pltpu.sync_copy(acc_sh.at[pl.ds(r0, RPT)],
                            out_hbm.at[c, pl.ds(r0, RPT)])

        @pl.when(sid == NS - 1)
        def _drain_last():
            pltpu.sync_copy(acc_sh.at[pl.ds(r0, RPT_LAST)],
                            out_hbm.at[c, pl.ds(r0, RPT_LAST)])


def kernel(x, edge_index, edge_weight, W0, b0, W1, b1, W2, b2, Wl, bl):
    dst = edge_index[0]
    src = edge_index[1]

    h = _mm_first(x, W0.astype(jnp.bfloat16), b0.reshape(NCHUNK, 1, CHUNK))
    y = _sc_scatter(h, src, dst, edge_weight)
    h = _mm_mid(y, W1.astype(jnp.bfloat16), b1.reshape(NCHUNK, 1, CHUNK))
    y = _sc_scatter(h, src, dst, edge_weight)
    h = _mm_mid(y, W2.astype(jnp.bfloat16), b2.reshape(NCHUNK, 1, CHUNK))
    y = _sc_scatter(h, src, dst, edge_weight)
    return _mm_last(y, Wl.astype(jnp.bfloat16), bl.reshape(1, D_OUT))


# speedup vs baseline: 7.0909x; 1.0666x over previous
import functools

import jax
import jax.numpy as jnp
from jax import lax
from jax.experimental import pallas as pl
from jax.experimental.pallas import tpu as pltpu
from jax.experimental.pallas import tpu_sc as plsc

N = 10000
E = 160000
D_IN = 256
D_HID = 512
D_OUT = 256

CHUNK = 128
NCHUNK = D_HID // CHUNK
NC = 2
NS = 16
CHUNKS_PER_CORE = NCHUNK // NC
EPT = E // NS
KB = 80
NBATCH = EPT // KB
RING = 4
LOOK = 2
NB_MAIN = NBATCH - 1
ZROWS = 40
RPT = 640
RPT_LAST = N - 15 * RPT
MB = 1000


def _mm_first_body(x_ref, w_ref, b_ref, o_ref):
    xb = x_ref[...].astype(jnp.bfloat16)
    for c in range(NCHUNK):
        acc = lax.dot_general(xb, w_ref[pl.ds(c * CHUNK, CHUNK), :],
                              (((1,), (1,)), ((), ())),
                              preferred_element_type=jnp.float32)
        o_ref[c] = acc + b_ref[c, 0][None, :]


def _mm_first(x, W, b4):
    return pl.pallas_call(
        _mm_first_body,
        grid=(N // MB,),
        in_specs=[
            pl.BlockSpec((MB, D_IN), lambda m: (m, 0)),
            pl.BlockSpec((D_HID, D_IN), lambda m: (0, 0)),
            pl.BlockSpec((NCHUNK, 1, CHUNK), lambda m: (0, 0, 0)),
        ],
        out_specs=pl.BlockSpec((NCHUNK, MB, CHUNK), lambda m: (0, m, 0)),
        out_shape=jax.ShapeDtypeStruct((NCHUNK, N, CHUNK), jnp.float32),
    )(x, W, b4)


def _mm_mid_body(y_ref, w_ref, b_ref, o_ref):
    yks = [jnp.maximum(y_ref[kc], 0.0).astype(jnp.bfloat16)
           for kc in range(NCHUNK)]
    for c in range(NCHUNK):
        acc = jnp.zeros((MB, CHUNK), jnp.float32)
        for kc in range(NCHUNK):
            wk = w_ref[pl.ds(c * CHUNK, CHUNK), pl.ds(kc * CHUNK, CHUNK)]
            acc = acc + lax.dot_general(yks[kc], wk,
                                        (((1,), (1,)), ((), ())),
                                        preferred_element_type=jnp.float32)
        o_ref[c] = acc + b_ref[c, 0][None, :]


def _mm_mid(y, W, b4):
    return pl.pallas_call(
        _mm_mid_body,
        grid=(N // MB,),
        in_specs=[
            pl.BlockSpec((NCHUNK, MB, CHUNK), lambda m: (0, m, 0)),
            pl.BlockSpec((D_HID, D_HID), lambda m: (0, 0)),
            pl.BlockSpec((NCHUNK, 1, CHUNK), lambda m: (0, 0, 0)),
        ],
        out_specs=pl.BlockSpec((NCHUNK, MB, CHUNK), lambda m: (0, m, 0)),
        out_shape=jax.ShapeDtypeStruct((NCHUNK, N, CHUNK), jnp.float32),
    )(y, W, b4)


def _mm_last_body(y_ref, w_ref, b_ref, o_ref):
    acc = jnp.zeros((MB, D_OUT), jnp.float32)
    for kc in range(NCHUNK):
        yk = jnp.maximum(y_ref[kc], 0.0).astype(jnp.bfloat16)
        wk = w_ref[:, kc * CHUNK:(kc + 1) * CHUNK]
        acc = acc + lax.dot_general(yk, wk, (((1,), (1,)), ((), ())),
                                    preferred_element_type=jnp.float32)
    h = acc + b_ref[...]
    norm = jnp.sqrt(jnp.sum(h * h, axis=1, keepdims=True))
    o_ref[...] = h / jnp.maximum(norm, 1e-12)


def _mm_last(y, Wl, bl2):
    return pl.pallas_call(
        _mm_last_body,
        grid=(N // MB,),
        in_specs=[
            pl.BlockSpec((NCHUNK, MB, CHUNK), lambda m: (0, m, 0)),
            pl.BlockSpec((D_OUT, D_HID), lambda m: (0, 0)),
            pl.BlockSpec((1, D_OUT), lambda m: (0, 0)),
        ],
        out_specs=pl.BlockSpec((MB, D_OUT), lambda m: (m, 0)),
        out_shape=jax.ShapeDtypeStruct((N, D_OUT), jnp.float32),
    )(y, Wl, bl2)


_SC_MESH = plsc.VectorSubcoreMesh(core_axis_name="c", subcore_axis_name="s")


_SC_SCRATCH = (
    [pltpu.VMEM((KB,), jnp.int32) for _ in range(RING)]
    + [pltpu.VMEM((KB,), jnp.int32) for _ in range(RING)]
    + [pltpu.VMEM((KB,), jnp.float32) for _ in range(RING)]
    + [pltpu.VMEM((KB, CHUNK), jnp.float32) for _ in range(RING)]
    + [pltpu.VMEM((ZROWS, CHUNK), jnp.float32)]
    + [pltpu.VMEM_SHARED((N, CHUNK), jnp.float32)]
    + [pltpu.SemaphoreType.DMA for _ in range(4 * RING + 1)]
)


@functools.partial(
    pl.kernel,
    out_type=jax.ShapeDtypeStruct((NCHUNK, N, CHUNK), jnp.float32),
    mesh=_SC_MESH,
    scratch_types=_SC_SCRATCH,
)
def _sc_scatter(h_hbm, src_hbm, dst_hbm, w_hbm, out_hbm, *scr):
    it = iter(scr)
    srcb = [next(it) for _ in range(RING)]
    dstb = [next(it) for _ in range(RING)]
    wb = [next(it) for _ in range(RING)]
    rows = [next(it) for _ in range(RING)]
    zero_v = next(it)
    acc_sh = next(it)
    semI = [next(it) for _ in range(RING)]
    semD = [next(it) for _ in range(RING)]
    semG = [next(it) for _ in range(RING)]
    semS = [next(it) for _ in range(RING)]
    semZ = next(it)

    cid = lax.axis_index("c")
    sid = lax.axis_index("s")
    e0 = pl.multiple_of(sid * EPT, 8)
    r0 = pl.multiple_of(sid * RPT, 8)

    def _fetch_srcw(b, k):
        off = e0 + pl.multiple_of(b * KB, 8)
        pltpu.async_copy(src_hbm.at[pl.ds(off, KB)], srcb[k], semI[k])
        pltpu.async_copy(w_hbm.at[pl.ds(off, KB)], wb[k], semI[k])

    def _wait_srcw(k):
        pltpu.make_async_copy(src_hbm.at[pl.ds(e0, KB)], srcb[k],
                              semI[k]).wait()
        pltpu.make_async_copy(w_hbm.at[pl.ds(e0, KB)], wb[k],
                              semI[k]).wait()

    def _fetch_dst(b, k):
        off = e0 + pl.multiple_of(b * KB, 8)
        pltpu.async_copy(dst_hbm.at[pl.ds(off, KB)], dstb[k], semD[k])

    def _wait_dst(k):
        pltpu.make_async_copy(dst_hbm.at[pl.ds(e0, KB)], dstb[k],
                              semD[k]).wait()

    def _gather(c, k):
        pltpu.async_copy(h_hbm.at[c].at[srcb[k]], rows[k], semG[k])

    def _wait_gather(c, k):
        pltpu.make_async_copy(h_hbm.at[c].at[srcb[k]], rows[k],
                              semG[k]).wait()

    def _scatter(k):
        pltpu.async_copy(rows[k], acc_sh.at[dstb[k]], semS[k], add=True)

    def _wait_scatter(k):
        pltpu.make_async_copy(rows[k], acc_sh.at[dstb[k]], semS[k]).wait()

    def _visit(b, k, c, refill):
        _wait_gather(c, k)

        def _grp(g, carry2):
            w16 = wb[k][pl.ds(g * 16, 16)]
            for j2 in range(16):
                wv = jnp.full((16,), w16[j2], jnp.float32)
                r = g * 16 + j2
                for f in range(CHUNK // 16):
                    sl = pl.ds(f * 16, 16)
                    rows[k][r, sl] = rows[k][r, sl] * wv
            return carry2
        lax.fori_loop(0, KB // 16, _grp, 0)

        _wait_dst(k)
        _scatter(k)

        if refill:
            k2 = (k + LOOK) % RING
            bt = b + LOOK

            @pl.when(bt < NBATCH)
            def _refill():
                @pl.when(b >= LOOK)
                def _protect():
                    _wait_scatter(k2)
                _fetch_dst(bt, k2)
                _wait_srcw(k2)
                _gather(c, k2)

            @pl.when(b + RING < NBATCH)
            def _prefetch():
                _fetch_srcw(b + RING, k)

    for r in range(ZROWS):
        for f in range(CHUNK // 16):
            zero_v[r, pl.ds(f * 16, 16)] = jnp.zeros((16,), jnp.float32)

    for i in range(CHUNKS_PER_CORE):
        c = cid * CHUNKS_PER_CORE + i

        for k in range(RING):
            _fetch_srcw(k, k)
        for k in range(LOOK):
            _fetch_dst(k, k)
        for k in range(LOOK):
            _wait_srcw(k)
            _gather(c, k)

        @pl.when(sid < NS - 1)
        def _zero():
            for q in range(RPT // ZROWS):
                pltpu.async_copy(zero_v,
                                 acc_sh.at[pl.ds(r0 + q * ZROWS, ZROWS)],
                                 semZ)
            for q in range(RPT // ZROWS):
                pltpu.make_async_copy(zero_v,
                                      acc_sh.at[pl.ds(r0, ZROWS)],
                                      semZ).wait()

        @pl.when(sid == NS - 1)
        def _zero_last():
            for q in range(RPT_LAST // ZROWS):
                pltpu.async_copy(zero_v,
                                 acc_sh.at[pl.ds(r0 + q * ZROWS, ZROWS)],
                                 semZ)
            for q in range(RPT_LAST // ZROWS):
                pltpu.make_async_copy(zero_v,
                                      acc_sh.at[pl.ds(r0, ZROWS)],
                                      semZ).wait()
        plsc.subcore_barrier()

        def _iter(it, carry):
            b_base = it * RING
            for k in range(RING):
                _visit(b_base + k, k, c, refill=True)
            return carry
        lax.fori_loop(0, NB_MAIN // RING, _iter, 0)

        _visit(NB_MAIN, NB_MAIN % RING, c, refill=False)

        for k in range(RING):
            _wait_scatter(k)
        plsc.subcore_barrier()

        @pl.when(sid < NS - 1)
        def _drain():
            pltpu.sync_copy(acc_sh.at[pl.ds(r0, RPT)],
                            out_hbm.at[c, pl.ds(r0, RPT)])

        @pl.when(sid == NS - 1)
        def _drain_last():
            pltpu.sync_copy(acc_sh.at[pl.ds(r0, RPT_LAST)],
                            out_hbm.at[c, pl.ds(r0, RPT_LAST)])


def kernel(x, edge_index, edge_weight, W0, b0, W1, b1, W2, b2, Wl, bl):
    dst = edge_index[0]
    src = edge_index[1]

    h = _mm_first(x, W0.astype(jnp.bfloat16), b0.reshape(NCHUNK, 1, CHUNK))
    y = _sc_scatter(h, src, dst, edge_weight)
    h = _mm_mid(y, W1.astype(jnp.bfloat16), b1.reshape(NCHUNK, 1, CHUNK))
    y = _sc_scatter(h, src, dst, edge_weight)
    h = _mm_mid(y, W2.astype(jnp.bfloat16), b2.reshape(NCHUNK, 1, CHUNK))
    y = _sc_scatter(h, src, dst, edge_weight)
    return _mm_last(y, Wl.astype(jnp.bfloat16), bl.reshape(1, D_OUT))


# speedup vs baseline: 7.5010x; 1.0578x over previous
import functools

import jax
import jax.numpy as jnp
from jax import lax
from jax.experimental import pallas as pl
from jax.experimental.pallas import tpu as pltpu
from jax.experimental.pallas import tpu_sc as plsc

N = 10000
E = 160000
D_IN = 256
D_HID = 512
D_OUT = 256

CHUNK = 128
NCHUNK = D_HID // CHUNK
NC = 2
NS = 16
CHUNKS_PER_CORE = NCHUNK // NC
EPT = E // NS
KB = 80
NBATCH = EPT // KB
RING = 4
LOOK = 3
NB_MAIN = NBATCH - 1
ZROWS = 40
RPT = 640
RPT_LAST = N - 15 * RPT
MB = 1000


def _mm_first_body(x_ref, w_ref, b_ref, o_ref):
    xb = x_ref[...].astype(jnp.bfloat16)
    for c in range(NCHUNK):
        acc = lax.dot_general(xb, w_ref[pl.ds(c * CHUNK, CHUNK), :],
                              (((1,), (1,)), ((), ())),
                              preferred_element_type=jnp.float32)
        o_ref[c] = acc + b_ref[c, 0][None, :]


def _mm_first(x, W, b4):
    return pl.pallas_call(
        _mm_first_body,
        grid=(N // MB,),
        in_specs=[
            pl.BlockSpec((MB, D_IN), lambda m: (m, 0)),
            pl.BlockSpec((D_HID, D_IN), lambda m: (0, 0)),
            pl.BlockSpec((NCHUNK, 1, CHUNK), lambda m: (0, 0, 0)),
        ],
        out_specs=pl.BlockSpec((NCHUNK, MB, CHUNK), lambda m: (0, m, 0)),
        out_shape=jax.ShapeDtypeStruct((NCHUNK, N, CHUNK), jnp.float32),
    )(x, W, b4)


def _mm_mid_body(y_ref, w_ref, b_ref, o_ref):
    yks = [jnp.maximum(y_ref[kc], 0.0).astype(jnp.bfloat16)
           for kc in range(NCHUNK)]
    for c in range(NCHUNK):
        acc = jnp.zeros((MB, CHUNK), jnp.float32)
        for kc in range(NCHUNK):
            wk = w_ref[pl.ds(c * CHUNK, CHUNK), pl.ds(kc * CHUNK, CHUNK)]
            acc = acc + lax.dot_general(yks[kc], wk,
                                        (((1,), (1,)), ((), ())),
                                        preferred_element_type=jnp.float32)
        o_ref[c] = acc + b_ref[c, 0][None, :]


def _mm_mid(y, W, b4):
    return pl.pallas_call(
        _mm_mid_body,
        grid=(N // MB,),
        in_specs=[
            pl.BlockSpec((NCHUNK, MB, CHUNK), lambda m: (0, m, 0)),
            pl.BlockSpec((D_HID, D_HID), lambda m: (0, 0)),
            pl.BlockSpec((NCHUNK, 1, CHUNK), lambda m: (0, 0, 0)),
        ],
        out_specs=pl.BlockSpec((NCHUNK, MB, CHUNK), lambda m: (0, m, 0)),
        out_shape=jax.ShapeDtypeStruct((NCHUNK, N, CHUNK), jnp.float32),
    )(y, W, b4)


def _mm_last_body(y_ref, w_ref, b_ref, o_ref):
    acc = jnp.zeros((MB, D_OUT), jnp.float32)
    for kc in range(NCHUNK):
        yk = jnp.maximum(y_ref[kc], 0.0).astype(jnp.bfloat16)
        wk = w_ref[:, kc * CHUNK:(kc + 1) * CHUNK]
        acc = acc + lax.dot_general(yk, wk, (((1,), (1,)), ((), ())),
                                    preferred_element_type=jnp.float32)
    h = acc + b_ref[...]
    norm = jnp.sqrt(jnp.sum(h * h, axis=1, keepdims=True))
    o_ref[...] = h / jnp.maximum(norm, 1e-12)


def _mm_last(y, Wl, bl2):
    return pl.pallas_call(
        _mm_last_body,
        grid=(N // MB,),
        in_specs=[
            pl.BlockSpec((NCHUNK, MB, CHUNK), lambda m: (0, m, 0)),
            pl.BlockSpec((D_OUT, D_HID), lambda m: (0, 0)),
            pl.BlockSpec((1, D_OUT), lambda m: (0, 0)),
        ],
        out_specs=pl.BlockSpec((MB, D_OUT), lambda m: (m, 0)),
        out_shape=jax.ShapeDtypeStruct((N, D_OUT), jnp.float32),
    )(y, Wl, bl2)


_SC_MESH = plsc.VectorSubcoreMesh(core_axis_name="c", subcore_axis_name="s")


_SC_SCRATCH = (
    [pltpu.VMEM((KB,), jnp.int32) for _ in range(RING)]
    + [pltpu.VMEM((KB,), jnp.int32) for _ in range(RING)]
    + [pltpu.VMEM((KB,), jnp.float32) for _ in range(RING)]
    + [pltpu.VMEM((KB, CHUNK), jnp.float32) for _ in range(RING)]
    + [pltpu.VMEM((ZROWS, CHUNK), jnp.float32)]
    + [pltpu.VMEM_SHARED((N, CHUNK), jnp.float32)]
    + [pltpu.SemaphoreType.DMA for _ in range(4 * RING + 1)]
)


@functools.partial(
    pl.kernel,
    out_type=jax.ShapeDtypeStruct((NCHUNK, N, CHUNK), jnp.float32),
    mesh=_SC_MESH,
    scratch_types=_SC_SCRATCH,
)
def _sc_scatter(h_hbm, src_hbm, dst_hbm, w_hbm, out_hbm, *scr):
    it = iter(scr)
    srcb = [next(it) for _ in range(RING)]
    dstb = [next(it) for _ in range(RING)]
    wb = [next(it) for _ in range(RING)]
    rows = [next(it) for _ in range(RING)]
    zero_v = next(it)
    acc_sh = next(it)
    semI = [next(it) for _ in range(RING)]
    semD = [next(it) for _ in range(RING)]
    semG = [next(it) for _ in range(RING)]
    semS = [next(it) for _ in range(RING)]
    semZ = next(it)

    cid = lax.axis_index("c")
    sid = lax.axis_index("s")
    e0 = pl.multiple_of(sid * EPT, 8)
    r0 = pl.multiple_of(sid * RPT, 8)

    def _fetch_srcw(b, k):
        off = e0 + pl.multiple_of(b * KB, 8)
        pltpu.async_copy(src_hbm.at[pl.ds(off, KB)], srcb[k], semI[k])
        pltpu.async_copy(w_hbm.at[pl.ds(off, KB)], wb[k], semI[k])

    def _wait_srcw(k):
        pltpu.make_async_copy(src_hbm.at[pl.ds(e0, KB)], srcb[k],
                              semI[k]).wait()
        pltpu.make_async_copy(w_hbm.at[pl.ds(e0, KB)], wb[k],
                              semI[k]).wait()

    def _fetch_dst(b, k):
        off = e0 + pl.multiple_of(b * KB, 8)
        pltpu.async_copy(dst_hbm.at[pl.ds(off, KB)], dstb[k], semD[k])

    def _wait_dst(k):
        pltpu.make_async_copy(dst_hbm.at[pl.ds(e0, KB)], dstb[k],
                              semD[k]).wait()

    def _gather(c, k):
        pltpu.async_copy(h_hbm.at[c].at[srcb[k]], rows[k], semG[k])

    def _wait_gather(c, k):
        pltpu.make_async_copy(h_hbm.at[c].at[srcb[k]], rows[k],
                              semG[k]).wait()

    def _scatter(k):
        pltpu.async_copy(rows[k], acc_sh.at[dstb[k]], semS[k], add=True)

    def _wait_scatter(k):
        pltpu.make_async_copy(rows[k], acc_sh.at[dstb[k]], semS[k]).wait()

    def _visit(b, k, c, refill):
        _wait_gather(c, k)

        def _grp(g, carry2):
            w16 = wb[k][pl.ds(g * 16, 16)]
            for j2 in range(16):
                wv = jnp.full((16,), w16[j2], jnp.float32)
                r = g * 16 + j2
                for f in range(CHUNK // 16):
                    sl = pl.ds(f * 16, 16)
                    rows[k][r, sl] = rows[k][r, sl] * wv
            return carry2
        lax.fori_loop(0, KB // 16, _grp, 0)

        _wait_dst(k)
        _scatter(k)

        if refill:
            k2 = (k + LOOK) % RING
            bt = b + LOOK

            @pl.when(bt < NBATCH)
            def _refill():
                @pl.when(b >= RING - LOOK)
                def _protect():
                    _wait_scatter(k2)
                _fetch_dst(bt, k2)
                _wait_srcw(k2)
                _gather(c, k2)

            @pl.when(b + RING < NBATCH)
            def _prefetch():
                _fetch_srcw(b + RING, k)

    for r in range(ZROWS):
        for f in range(CHUNK // 16):
            zero_v[r, pl.ds(f * 16, 16)] = jnp.zeros((16,), jnp.float32)

    for i in range(CHUNKS_PER_CORE):
        c = cid * CHUNKS_PER_CORE + i

        for k in range(RING):
            _fetch_srcw(k, k)
        for k in range(LOOK):
            _fetch_dst(k, k)
        for k in range(LOOK):
            _wait_srcw(k)
            _gather(c, k)

        @pl.when(sid < NS - 1)
        def _zero():
            for q in range(RPT // ZROWS):
                pltpu.async_copy(zero_v,
                                 acc_sh.at[pl.ds(r0 + q * ZROWS, ZROWS)],
                                 semZ)
            for q in range(RPT // ZROWS):
                pltpu.make_async_copy(zero_v,
                                      acc_sh.at[pl.ds(r0, ZROWS)],
                                      semZ).wait()

        @pl.when(sid == NS - 1)
        def _zero_last():
            for q in range(RPT_LAST // ZROWS):
                pltpu.async_copy(zero_v,
                                 acc_sh.at[pl.ds(r0 + q * ZROWS, ZROWS)],
                                 semZ)
            for q in range(RPT_LAST // ZROWS):
                pltpu.make_async_copy(zero_v,
                                      acc_sh.at[pl.ds(r0, ZROWS)],
                                      semZ).wait()
        plsc.subcore_barrier()

        def _iter(it, carry):
            b_base = it * RING
            for k in range(RING):
                _visit(b_base + k, k, c, refill=True)
            return carry
        lax.fori_loop(0, NB_MAIN // RING, _iter, 0)

        _visit(NB_MAIN, NB_MAIN % RING, c, refill=False)

        for k in range(RING):
            _wait_scatter(k)
        plsc.subcore_barrier()

        @pl.when(sid < NS - 1)
        def _drain():
            pltpu.sync_copy(acc_sh.at[pl.ds(r0, RPT)],
                            out_hbm.at[c, pl.ds(r0, RPT)])

        @pl.when(sid == NS - 1)
        def _drain_last():
            pltpu.sync_copy(acc_sh.at[pl.ds(r0, RPT_LAST)],
                            out_hbm.at[c, pl.ds(r0, RPT_LAST)])


def kernel(x, edge_index, edge_weight, W0, b0, W1, b1, W2, b2, Wl, bl):
    dst = edge_index[0]
    src = edge_index[1]

    h = _mm_first(x, W0.astype(jnp.bfloat16), b0.reshape(NCHUNK, 1, CHUNK))
    y = _sc_scatter(h, src, dst, edge_weight)
    h = _mm_mid(y, W1.astype(jnp.bfloat16), b1.reshape(NCHUNK, 1, CHUNK))
    y = _sc_scatter(h, src, dst, edge_weight)
    h = _mm_mid(y, W2.astype(jnp.bfloat16), b2.reshape(NCHUNK, 1, CHUNK))
    y = _sc_scatter(h, src, dst, edge_weight)
    return _mm_last(y, Wl.astype(jnp.bfloat16), bl.reshape(1, D_OUT))


# speedup vs baseline: 7.5110x; 1.0013x over previous
import functools

import jax
import jax.numpy as jnp
from jax import lax
from jax.experimental import pallas as pl
from jax.experimental.pallas import tpu as pltpu
from jax.experimental.pallas import tpu_sc as plsc

N = 10000
E = 160000
D_IN = 256
D_HID = 512
D_OUT = 256

CHUNK = 128
NCHUNK = D_HID // CHUNK
NC = 2
NS = 16
CHUNKS_PER_CORE = NCHUNK // NC
EPT = E // NS
KB = 80
NBATCH = EPT // KB
RING = 4
LOOK = 3
NB_MAIN = NBATCH - 1
ZROWS = 40
RPT = 640
RPT_LAST = N - 15 * RPT
MB = 1000


def _mm_first_body(x_ref, w_ref, b_ref, o_ref):
    xb = x_ref[...].astype(jnp.bfloat16)
    for c in range(NCHUNK):
        acc = lax.dot_general(xb, w_ref[pl.ds(c * CHUNK, CHUNK), :],
                              (((1,), (1,)), ((), ())),
                              preferred_element_type=jnp.float32)
        o_ref[c] = acc + b_ref[c, 0][None, :]


def _mm_first(x, W, b4):
    return pl.pallas_call(
        _mm_first_body,
        grid=(N // MB,),
        in_specs=[
            pl.BlockSpec((MB, D_IN), lambda m: (m, 0)),
            pl.BlockSpec((D_HID, D_IN), lambda m: (0, 0)),
            pl.BlockSpec((NCHUNK, 1, CHUNK), lambda m: (0, 0, 0)),
        ],
        out_specs=pl.BlockSpec((NCHUNK, MB, CHUNK), lambda m: (0, m, 0)),
        out_shape=jax.ShapeDtypeStruct((NCHUNK, N, CHUNK), jnp.float32),
    )(x, W, b4)


def _mm_mid_body(y_ref, w_ref, b_ref, o_ref):
    yks = [jnp.maximum(y_ref[kc], 0.0).astype(jnp.bfloat16)
           for kc in range(NCHUNK)]
    for c in range(NCHUNK):
        acc = jnp.zeros((MB, CHUNK), jnp.float32)
        for kc in range(NCHUNK):
            wk = w_ref[pl.ds(c * CHUNK, CHUNK), pl.ds(kc * CHUNK, CHUNK)]
            acc = acc + lax.dot_general(yks[kc], wk,
                                        (((1,), (1,)), ((), ())),
                                        preferred_element_type=jnp.float32)
        o_ref[c] = acc + b_ref[c, 0][None, :]


def _mm_mid(y, W, b4):
    return pl.pallas_call(
        _mm_mid_body,
        grid=(N // MB,),
        in_specs=[
            pl.BlockSpec((NCHUNK, MB, CHUNK), lambda m: (0, m, 0)),
            pl.BlockSpec((D_HID, D_HID), lambda m: (0, 0)),
            pl.BlockSpec((NCHUNK, 1, CHUNK), lambda m: (0, 0, 0)),
        ],
        out_specs=pl.BlockSpec((NCHUNK, MB, CHUNK), lambda m: (0, m, 0)),
        out_shape=jax.ShapeDtypeStruct((NCHUNK, N, CHUNK), jnp.float32),
    )(y, W, b4)


def _mm_last_body(y_ref, w_ref, b_ref, o_ref):
    acc = jnp.zeros((MB, D_OUT), jnp.float32)
    for kc in range(NCHUNK):
        yk = jnp.maximum(y_ref[kc], 0.0).astype(jnp.bfloat16)
        wk = w_ref[:, kc * CHUNK:(kc + 1) * CHUNK]
        acc = acc + lax.dot_general(yk, wk, (((1,), (1,)), ((), ())),
                                    preferred_element_type=jnp.float32)
    h = acc + b_ref[...]
    norm = jnp.sqrt(jnp.sum(h * h, axis=1, keepdims=True))
    o_ref[...] = h / jnp.maximum(norm, 1e-12)


def _mm_last(y, Wl, bl2):
    return pl.pallas_call(
        _mm_last_body,
        grid=(N // MB,),
        in_specs=[
            pl.BlockSpec((NCHUNK, MB, CHUNK), lambda m: (0, m, 0)),
            pl.BlockSpec((D_OUT, D_HID), lambda m: (0, 0)),
            pl.BlockSpec((1, D_OUT), lambda m: (0, 0)),
        ],
        out_specs=pl.BlockSpec((MB, D_OUT), lambda m: (m, 0)),
        out_shape=jax.ShapeDtypeStruct((N, D_OUT), jnp.float32),
    )(y, Wl, bl2)


_SC_MESH = plsc.VectorSubcoreMesh(core_axis_name="c", subcore_axis_name="s")


_SC_SCRATCH = (
    [pltpu.VMEM((KB,), jnp.int32) for _ in range(RING)]
    + [pltpu.VMEM((KB,), jnp.int32) for _ in range(RING)]
    + [pltpu.VMEM((KB,), jnp.float32) for _ in range(RING)]
    + [pltpu.VMEM((KB, CHUNK), jnp.float32) for _ in range(RING)]
    + [pltpu.VMEM((ZROWS, CHUNK), jnp.float32)]
    + [pltpu.VMEM_SHARED((N, CHUNK), jnp.float32)]
    + [pltpu.SemaphoreType.DMA for _ in range(4 * RING + 1)]
)


@functools.partial(
    pl.kernel,
    out_type=jax.ShapeDtypeStruct((NCHUNK, N, CHUNK), jnp.float32),
    mesh=_SC_MESH,
    scratch_types=_SC_SCRATCH,
)
def _sc_scatter(h_hbm, src_hbm, dst_hbm, w_hbm, out_hbm, *scr):
    it = iter(scr)
    srcb = [next(it) for _ in range(RING)]
    dstb = [next(it) for _ in range(RING)]
    wb = [next(it) for _ in range(RING)]
    rows = [next(it) for _ in range(RING)]
    zero_v = next(it)
    acc_sh = next(it)
    semI = [next(it) for _ in range(RING)]
    semD = [next(it) for _ in range(RING)]
    semG = [next(it) for _ in range(RING)]
    semS = [next(it) for _ in range(RING)]
    semZ = next(it)

    cid = lax.axis_index("c")
    sid = lax.axis_index("s")
    e0 = pl.multiple_of(sid * EPT, 8)
    r0 = pl.multiple_of(sid * RPT, 8)

    def _fetch_srcw(b, k):
        off = e0 + pl.multiple_of(b * KB, 8)
        pltpu.async_copy(src_hbm.at[pl.ds(off, KB)], srcb[k], semI[k])
        pltpu.async_copy(w_hbm.at[pl.ds(off, KB)], wb[k], semI[k])

    def _wait_srcw(k):
        pltpu.make_async_copy(src_hbm.at[pl.ds(e0, KB)], srcb[k],
                              semI[k]).wait()
        pltpu.make_async_copy(w_hbm.at[pl.ds(e0, KB)], wb[k],
                              semI[k]).wait()

    def _fetch_dst(b, k):
        off = e0 + pl.multiple_of(b * KB, 8)
        pltpu.async_copy(dst_hbm.at[pl.ds(off, KB)], dstb[k], semD[k])

    def _wait_dst(k):
        pltpu.make_async_copy(dst_hbm.at[pl.ds(e0, KB)], dstb[k],
                              semD[k]).wait()

    HK = KB // 2

    def _gather(c, k):
        pltpu.async_copy(h_hbm.at[c].at[srcb[k].at[pl.ds(0, HK)]],
                         rows[k].at[pl.ds(0, HK)], semG[k])
        pltpu.async_copy(h_hbm.at[c].at[srcb[k].at[pl.ds(HK, HK)]],
                         rows[k].at[pl.ds(HK, HK)], semG[k])

    def _wait_gather(c, k):
        pltpu.make_async_copy(h_hbm.at[c].at[srcb[k].at[pl.ds(0, HK)]],
                              rows[k].at[pl.ds(0, HK)], semG[k]).wait()
        pltpu.make_async_copy(h_hbm.at[c].at[srcb[k].at[pl.ds(HK, HK)]],
                              rows[k].at[pl.ds(HK, HK)], semG[k]).wait()

    def _scatter(k):
        pltpu.async_copy(rows[k], acc_sh.at[dstb[k]], semS[k], add=True)

    def _wait_scatter(k):
        pltpu.make_async_copy(rows[k], acc_sh.at[dstb[k]], semS[k]).wait()

    def _visit(b, k, c, refill):
        _wait_gather(c, k)

        def _grp(g, carry2):
            w16 = wb[k][pl.ds(g * 16, 16)]
            for j2 in range(16):
                wv = jnp.full((16,), w16[j2], jnp.float32)
                r = g * 16 + j2
                for f in range(CHUNK // 16):
                    sl = pl.ds(f * 16, 16)
                    rows[k][r, sl] = rows[k][r, sl] * wv
            return carry2
        lax.fori_loop(0, KB // 16, _grp, 0)

        _wait_dst(k)
        _scatter(k)

        if refill:
            k2 = (k + LOOK) % RING
            bt = b + LOOK

            @pl.when(bt < NBATCH)
            def _refill():
                @pl.when(b >= RING - LOOK)
                def _protect():
                    _wait_scatter(k2)
                _fetch_dst(bt, k2)
                _wait_srcw(k2)
                _gather(c, k2)

            @pl.when(b + RING < NBATCH)
            def _prefetch():
                _fetch_srcw(b + RING, k)

    for r in range(ZROWS):
        for f in range(CHUNK // 16):
            zero_v[r, pl.ds(f * 16, 16)] = jnp.zeros((16,), jnp.float32)

    for i in range(CHUNKS_PER_CORE):
        c = cid * CHUNKS_PER_CORE + i

        for k in range(RING):
            _fetch_srcw(k, k)
        for k in range(LOOK):
            _fetch_dst(k, k)
        for k in range(LOOK):
            _wait_srcw(k)
            _gather(c, k)

        @pl.when(sid < NS - 1)
        def _zero():
            for q in range(RPT // ZROWS):
                pltpu.async_copy(zero_v,
                                 acc_sh.at[pl.ds(r0 + q * ZROWS, ZROWS)],
                                 semZ)
            for q in range(RPT // ZROWS):
                pltpu.make_async_copy(zero_v,
                                      acc_sh.at[pl.ds(r0, ZROWS)],
                                      semZ).wait()

        @pl.when(sid == NS - 1)
        def _zero_last():
            for q in range(RPT_LAST // ZROWS):
                pltpu.async_copy(zero_v,
                                 acc_sh.at[pl.ds(r0 + q * ZROWS, ZROWS)],
                                 semZ)
            for q in range(RPT_LAST // ZROWS):
                pltpu.make_async_copy(zero_v,
                                      acc_sh.at[pl.ds(r0, ZROWS)],
                                      semZ).wait()
        plsc.subcore_barrier()

        def _iter(it, carry):
            b_base = it * RING
            for k in range(RING):
                _visit(b_base + k, k, c, refill=True)
            return carry
        lax.fori_loop(0, NB_MAIN // RING, _iter, 0)

        _visit(NB_MAIN, NB_MAIN % RING, c, refill=False)

        for k in range(RING):
            _wait_scatter(k)
        plsc.subcore_barrier()

        @pl.when(sid < NS - 1)
        def _drain():
            pltpu.sync_copy(acc_sh.at[pl.ds(r0, RPT)],
                            out_hbm.at[c, pl.ds(r0, RPT)])

        @pl.when(sid == NS - 1)
        def _drain_last():
            pltpu.sync_copy(acc_sh.at[pl.ds(r0, RPT_LAST)],
                            out_hbm.at[c, pl.ds(r0, RPT_LAST)])


def kernel(x, edge_index, edge_weight, W0, b0, W1, b1, W2, b2, Wl, bl):
    dst = edge_index[0]
    src = edge_index[1]

    h = _mm_first(x, W0.astype(jnp.bfloat16), b0.reshape(NCHUNK, 1, CHUNK))
    y = _sc_scatter(h, src, dst, edge_weight)
    h = _mm_mid(y, W1.astype(jnp.bfloat16), b1.reshape(NCHUNK, 1, CHUNK))
    y = _sc_scatter(h, src, dst, edge_weight)
    h = _mm_mid(y, W2.astype(jnp.bfloat16), b2.reshape(NCHUNK, 1, CHUNK))
    y = _sc_scatter(h, src, dst, edge_weight)
    return _mm_last(y, Wl.astype(jnp.bfloat16), bl.reshape(1, D_OUT))
